# baseline (speedup 1.0000x reference)
import functools

import jax
import jax.numpy as jnp
import numpy as np
from jax.experimental import pallas as pl
from jax.experimental.pallas import tpu as pltpu


def _np_threefry2x32(k0, k1, x0, x1):
    x0 = np.asarray(x0, np.uint32).copy()
    x1 = np.asarray(x1, np.uint32).copy()
    ks0 = np.uint32(k0)
    ks1 = np.uint32(k1)
    ks2 = np.uint32(ks0 ^ ks1 ^ np.uint32(0x1BD11BDA))

    def rotl(v, d):
        return ((v << np.uint32(d)) | (v >> np.uint32(32 - d))).astype(np.uint32)

    x0 = (x0 + ks0).astype(np.uint32)
    x1 = (x1 + ks1).astype(np.uint32)
    ks = [ks0, ks1, ks2]
    rots = ((13, 15, 26, 6), (17, 29, 16, 24))
    inject = [(1, 2), (2, 0), (0, 1), (1, 2), (2, 0)]
    for i in range(5):
        for r in rots[i % 2]:
            x0 = (x0 + x1).astype(np.uint32)
            x1 = rotl(x1, r)
            x1 = x1 ^ x0
        a, b = inject[i]
        x0 = (x0 + ks[a]).astype(np.uint32)
        x1 = (x1 + ks[b] + np.uint32(i + 1)).astype(np.uint32)
    return x0, x1


@functools.lru_cache(maxsize=4)
def _keep_mask_i32(nnz: int):
    i = np.arange(nnz, dtype=np.uint32)
    y0, y1 = _np_threefry2x32(0, 42, np.zeros(nnz, np.uint32), i)
    bits = y0 ^ y1
    u = ((bits >> np.uint32(9)) | np.uint32(0x3F800000)).view(np.float32)
    u = u - np.float32(1.0)
    drop = (u + np.float32(0.2)) > np.float32(1.0)
    return np.where(drop, 0.0, 1.0).astype(np.float32)


def _select_block(v_ref, m_ref, i_ref, o_ref, oi_ref, sem, *, grid: int):
    pid = pl.program_id(0)

    @pl.when(pid == 0)
    def _():
        pltpu.make_async_copy(i_ref, oi_ref, sem).start()

    o_ref[...] = v_ref[...] * m_ref[...].astype(jnp.float32)

    @pl.when(pid == grid - 1)
    def _():
        pltpu.make_async_copy(i_ref, oi_ref, sem).wait()


def kernel(adj_indices, adj_values):
    nnz = adj_values.shape[0]
    block = 1146880
    grid = pl.cdiv(nnz, block)
    mask = jnp.asarray(_keep_mask_i32(nnz)).astype(jnp.bfloat16)
    val, idx = pl.pallas_call(
        functools.partial(_select_block, grid=grid),
        grid=(grid,),
        in_specs=[pl.BlockSpec((block,), lambda i: (i,)),
                  pl.BlockSpec((block,), lambda i: (i,)),
                  pl.BlockSpec(memory_space=pl.ANY)],
        out_specs=[pl.BlockSpec((block,), lambda i: (i,)),
                   pl.BlockSpec(memory_space=pl.ANY)],
        out_shape=[jax.ShapeDtypeStruct((nnz,), jnp.float32),
                   jax.ShapeDtypeStruct((2, nnz), adj_indices.dtype)],
        scratch_shapes=[pltpu.SemaphoreType.DMA],
    )(adj_values, mask, adj_indices)
    return (idx, val)


# speedup vs baseline: 27.2521x; 27.2521x over previous
import functools

import jax
import jax.numpy as jnp
import numpy as np
from jax.experimental import pallas as pl


def _np_threefry2x32(k0, k1, x0, x1):
    x0 = np.asarray(x0, np.uint32).copy()
    x1 = np.asarray(x1, np.uint32).copy()
    ks0 = np.uint32(k0)
    ks1 = np.uint32(k1)
    ks2 = np.uint32(ks0 ^ ks1 ^ np.uint32(0x1BD11BDA))

    def rotl(v, d):
        return ((v << np.uint32(d)) | (v >> np.uint32(32 - d))).astype(np.uint32)

    x0 = (x0 + ks0).astype(np.uint32)
    x1 = (x1 + ks1).astype(np.uint32)
    ks = [ks0, ks1, ks2]
    rots = ((13, 15, 26, 6), (17, 29, 16, 24))
    inject = [(1, 2), (2, 0), (0, 1), (1, 2), (2, 0)]
    for i in range(5):
        for r in rots[i % 2]:
            x0 = (x0 + x1).astype(np.uint32)
            x1 = rotl(x1, r)
            x1 = x1 ^ x0
        a, b = inject[i]
        x0 = (x0 + ks[a]).astype(np.uint32)
        x1 = (x1 + ks[b] + np.uint32(i + 1)).astype(np.uint32)
    return x0, x1


@functools.lru_cache(maxsize=4)
def _keep_mask_i32(nnz: int):
    i = np.arange(nnz, dtype=np.uint32)
    y0, y1 = _np_threefry2x32(0, 42, np.zeros(nnz, np.uint32), i)
    bits = y0 ^ y1
    u = ((bits >> np.uint32(9)) | np.uint32(0x3F800000)).view(np.float32)
    u = u - np.float32(1.0)
    drop = (u + np.float32(0.2)) > np.float32(1.0)
    return np.where(drop, 0.0, 1.0).astype(np.float32)


def _select_block(v_ref, m_ref, i_ref, o_ref, oi_ref):
    o_ref[...] = v_ref[...] * m_ref[...].astype(jnp.float32)
    oi_ref[...] = i_ref[...]


def kernel(adj_indices, adj_values):
    nnz = adj_values.shape[0]
    block = 1146880
    grid = pl.cdiv(nnz, block)
    mask = jnp.asarray(_keep_mask_i32(nnz)).astype(jnp.bfloat16)
    val, idx = pl.pallas_call(
        _select_block,
        grid=(grid,),
        in_specs=[pl.BlockSpec((block,), lambda i: (i,)),
                  pl.BlockSpec((block,), lambda i: (i,)),
                  pl.BlockSpec((2, block), lambda i: (0, i))],
        out_specs=[pl.BlockSpec((block,), lambda i: (i,)),
                   pl.BlockSpec((2, block), lambda i: (0, i))],
        out_shape=[jax.ShapeDtypeStruct((nnz,), jnp.float32),
                   jax.ShapeDtypeStruct((2, nnz), adj_indices.dtype)],
    )(adj_values, mask, adj_indices)
    return (idx, val)


# speedup vs baseline: 27.4149x; 1.0060x over previous
import functools

import jax
import jax.numpy as jnp
import numpy as np
from jax.experimental import pallas as pl


def _np_threefry2x32(k0, k1, x0, x1):
    x0 = np.asarray(x0, np.uint32).copy()
    x1 = np.asarray(x1, np.uint32).copy()
    ks0 = np.uint32(k0)
    ks1 = np.uint32(k1)
    ks2 = np.uint32(ks0 ^ ks1 ^ np.uint32(0x1BD11BDA))

    def rotl(v, d):
        return ((v << np.uint32(d)) | (v >> np.uint32(32 - d))).astype(np.uint32)

    x0 = (x0 + ks0).astype(np.uint32)
    x1 = (x1 + ks1).astype(np.uint32)
    ks = [ks0, ks1, ks2]
    rots = ((13, 15, 26, 6), (17, 29, 16, 24))
    inject = [(1, 2), (2, 0), (0, 1), (1, 2), (2, 0)]
    for i in range(5):
        for r in rots[i % 2]:
            x0 = (x0 + x1).astype(np.uint32)
            x1 = rotl(x1, r)
            x1 = x1 ^ x0
        a, b = inject[i]
        x0 = (x0 + ks[a]).astype(np.uint32)
        x1 = (x1 + ks[b] + np.uint32(i + 1)).astype(np.uint32)
    return x0, x1


@functools.lru_cache(maxsize=4)
def _keep_mask(nnz: int):
    i = np.arange(nnz, dtype=np.uint32)
    y0, y1 = _np_threefry2x32(0, 42, np.zeros(nnz, np.uint32), i)
    bits = y0 ^ y1
    u = ((bits >> np.uint32(9)) | np.uint32(0x3F800000)).view(np.float32)
    u = u - np.float32(1.0)
    drop = (u + np.float32(0.2)) > np.float32(1.0)
    return np.where(drop, 0.0, 1.0).astype(np.float32)


def _select_block(v_ref, m_ref, i_ref, o_ref, oi_ref):
    o_ref[...] = v_ref[...] * m_ref[...].astype(jnp.float32)
    oi_ref[...] = i_ref[...]


def kernel(adj_indices, adj_values):
    nnz = adj_values.shape[0]
    block = 1146880
    grid = pl.cdiv(nnz, block)
    mask = jnp.asarray(_keep_mask(nnz)).astype(jnp.bfloat16)
    val, idx = pl.pallas_call(
        _select_block,
        grid=(grid,),
        in_specs=[pl.BlockSpec((block,), lambda i: (i,)),
                  pl.BlockSpec((block,), lambda i: (i,)),
                  pl.BlockSpec((2, block), lambda i: (0, i))],
        out_specs=[pl.BlockSpec((block,), lambda i: (i,)),
                   pl.BlockSpec((2, block), lambda i: (0, i))],
        out_shape=[jax.ShapeDtypeStruct((nnz,), jnp.float32),
                   jax.ShapeDtypeStruct((2, nnz), adj_indices.dtype)],
    )(adj_values, mask, adj_indices)
    return (idx, val)
